# Initial kernel scaffold; baseline (speedup 1.0000x reference)
import jax
import jax.numpy as jnp
from jax.experimental import pallas as pl


def kernel(data):
    raise NotImplementedError("write your pallas kernel here")


# speedup vs baseline: 2.4984x; 2.4984x over previous
import numpy as np
import jax
import jax.numpy as jnp
from jax.experimental import pallas as pl


def _pts_table_t():
    vert_angles = np.radians(np.concatenate((
        np.linspace(4 + 1.0 / 3, -8 - 1.0 / 3, 40),
        np.linspace(-8 - 1.0 / 3 - 1.0 / 2, -24 - 1.0 / 3, 32))))
    hor_angles = np.radians(np.flip(np.arange(0, 360, 0.1728)) + 180)
    ray = np.array([1.0, 0, 0])
    vert_rotmat = np.array([[[np.cos(a), 0, -np.sin(a)], [0, 1, 0],
                             [np.sin(a), 0, np.cos(a)]] for a in vert_angles])
    hor_rotmat = np.array([[[np.cos(a), -np.sin(a), 0],
                            [np.sin(a), np.cos(a), 0],
                            [0, 0, 1]] for a in hor_angles])
    v = vert_rotmat @ ray
    pts = np.einsum('xij,yj->iyx', hor_rotmat, v)
    return jnp.asarray(pts.astype(np.float32))


_PTS_T = _pts_table_t()


def _xyz_kernel(data_ref, pts_ref, out_ref):
    dist = data_ref[0, 0]
    maskv = data_ref[0, 1]
    m = maskv >= 0.5
    zero = jnp.zeros((), dtype=dist.dtype)
    md = jnp.where(m, dist, zero)
    out_ref[0, 0] = md * pts_ref[0]
    out_ref[0, 1] = md * pts_ref[1]
    out_ref[0, 2] = md * pts_ref[2]
    out_ref[0, 3] = maskv


def kernel(data):
    b, c, ys, xs = data.shape
    pts = _PTS_T[:, :ys, :xs]
    return pl.pallas_call(
        _xyz_kernel,
        grid=(b,),
        in_specs=[
            pl.BlockSpec((1, c, ys, xs), lambda i: (i, 0, 0, 0)),
            pl.BlockSpec((3, ys, xs), lambda i: (0, 0, 0)),
        ],
        out_specs=pl.BlockSpec((1, 4, ys, xs), lambda i: (i, 0, 0, 0)),
        out_shape=jax.ShapeDtypeStruct((b, 4, ys, xs), data.dtype),
    )(data, pts)


# speedup vs baseline: 2.5009x; 1.0010x over previous
import numpy as np
import jax
import jax.numpy as jnp
from jax.experimental import pallas as pl


def _pts_table_t():
    vert_angles = np.radians(np.concatenate((
        np.linspace(4 + 1.0 / 3, -8 - 1.0 / 3, 40),
        np.linspace(-8 - 1.0 / 3 - 1.0 / 2, -24 - 1.0 / 3, 32))))
    hor_angles = np.radians(np.flip(np.arange(0, 360, 0.1728)) + 180)
    ray = np.array([1.0, 0, 0])
    vert_rotmat = np.array([[[np.cos(a), 0, -np.sin(a)], [0, 1, 0],
                             [np.sin(a), 0, np.cos(a)]] for a in vert_angles])
    hor_rotmat = np.array([[[np.cos(a), -np.sin(a), 0],
                            [np.sin(a), np.cos(a), 0],
                            [0, 0, 1]] for a in hor_angles])
    v = vert_rotmat @ ray
    pts = np.einsum('xij,yj->iyx', hor_rotmat, v)
    return pts.astype(np.float32)


_PTS_T = _pts_table_t()


def _xyz_kernel(data_ref, pts_ref, out_ref):
    dist = data_ref[0, 0]
    maskv = data_ref[0, 1]
    m = maskv >= 0.5
    zero = jnp.zeros((), dtype=dist.dtype)
    md = jnp.where(m, dist, zero)
    out_ref[0, 0] = md * pts_ref[0]
    out_ref[0, 1] = md * pts_ref[1]
    out_ref[0, 2] = md * pts_ref[2]
    out_ref[0, 3] = maskv


def kernel(data):
    b, c, ys, xs = data.shape
    pts = _PTS_T[:, :ys, :xs]
    return pl.pallas_call(
        _xyz_kernel,
        grid=(b,),
        in_specs=[
            pl.BlockSpec((1, c, ys, xs), lambda i: (i, 0, 0, 0)),
            pl.BlockSpec((3, ys, xs), lambda i: (0, 0, 0)),
        ],
        out_specs=pl.BlockSpec((1, 4, ys, xs), lambda i: (i, 0, 0, 0)),
        out_shape=jax.ShapeDtypeStruct((b, 4, ys, xs), data.dtype),
    )(data, pts)


# speedup vs baseline: 2.8372x; 1.1345x over previous
import numpy as np
import jax
import jax.numpy as jnp
from jax.experimental import pallas as pl


def _pts_table_t():
    vert_angles = np.radians(np.concatenate((
        np.linspace(4 + 1.0 / 3, -8 - 1.0 / 3, 40),
        np.linspace(-8 - 1.0 / 3 - 1.0 / 2, -24 - 1.0 / 3, 32))))
    hor_angles = np.radians(np.flip(np.arange(0, 360, 0.1728)) + 180)
    ray = np.array([1.0, 0, 0])
    vert_rotmat = np.array([[[np.cos(a), 0, -np.sin(a)], [0, 1, 0],
                             [np.sin(a), 0, np.cos(a)]] for a in vert_angles])
    hor_rotmat = np.array([[[np.cos(a), -np.sin(a), 0],
                            [np.sin(a), np.cos(a), 0],
                            [0, 0, 1]] for a in hor_angles])
    v = vert_rotmat @ ray
    pts = np.einsum('xij,yj->iyx', hor_rotmat, v)
    return pts.astype(np.float32)


_PTS_T = _pts_table_t()


_BB = 2


def _xyz_kernel(data_ref, pts_ref, out_ref):
    for i in range(_BB):
        dist = data_ref[i, 0]
        maskv = data_ref[i, 1]
        m = maskv >= 0.5
        zero = jnp.zeros((), dtype=dist.dtype)
        md = jnp.where(m, dist, zero)
        out_ref[i, 0] = md * pts_ref[0]
        out_ref[i, 1] = md * pts_ref[1]
        out_ref[i, 2] = md * pts_ref[2]
        out_ref[i, 3] = maskv


def kernel(data):
    b, c, ys, xs = data.shape
    pts = _PTS_T[:, :ys, :xs]
    return pl.pallas_call(
        _xyz_kernel,
        grid=(b // _BB,),
        in_specs=[
            pl.BlockSpec((_BB, c, ys, xs), lambda i: (i, 0, 0, 0)),
            pl.BlockSpec((3, ys, xs), lambda i: (0, 0, 0)),
        ],
        out_specs=pl.BlockSpec((_BB, 4, ys, xs), lambda i: (i, 0, 0, 0)),
        out_shape=jax.ShapeDtypeStruct((b, 4, ys, xs), data.dtype),
    )(data, pts)


# speedup vs baseline: 2.9539x; 1.0411x over previous
import numpy as np
import jax
import jax.numpy as jnp
from jax.experimental import pallas as pl


def _pts_table_t():
    vert_angles = np.radians(np.concatenate((
        np.linspace(4 + 1.0 / 3, -8 - 1.0 / 3, 40),
        np.linspace(-8 - 1.0 / 3 - 1.0 / 2, -24 - 1.0 / 3, 32))))
    hor_angles = np.radians(np.flip(np.arange(0, 360, 0.1728)) + 180)
    ray = np.array([1.0, 0, 0])
    vert_rotmat = np.array([[[np.cos(a), 0, -np.sin(a)], [0, 1, 0],
                             [np.sin(a), 0, np.cos(a)]] for a in vert_angles])
    hor_rotmat = np.array([[[np.cos(a), -np.sin(a), 0],
                            [np.sin(a), np.cos(a), 0],
                            [0, 0, 1]] for a in hor_angles])
    v = vert_rotmat @ ray
    pts = np.einsum('xij,yj->iyx', hor_rotmat, v)
    return pts.astype(np.float32)


_PTS_T = _pts_table_t()


_BB = 4


def _xyz_kernel(data_ref, pts_ref, out_ref):
    for i in range(_BB):
        dist = data_ref[i, 0]
        maskv = data_ref[i, 1]
        m = maskv >= 0.5
        zero = jnp.zeros((), dtype=dist.dtype)
        md = jnp.where(m, dist, zero)
        out_ref[i, 0] = md * pts_ref[0]
        out_ref[i, 1] = md * pts_ref[1]
        out_ref[i, 2] = md * pts_ref[2]
        out_ref[i, 3] = maskv


def kernel(data):
    b, c, ys, xs = data.shape
    pts = _PTS_T[:, :ys, :xs]
    return pl.pallas_call(
        _xyz_kernel,
        grid=(b // _BB,),
        in_specs=[
            pl.BlockSpec((_BB, c, ys, xs), lambda i: (i, 0, 0, 0)),
            pl.BlockSpec((3, ys, xs), lambda i: (0, 0, 0)),
        ],
        out_specs=pl.BlockSpec((_BB, 4, ys, xs), lambda i: (i, 0, 0, 0)),
        out_shape=jax.ShapeDtypeStruct((b, 4, ys, xs), data.dtype),
    )(data, pts)


# speedup vs baseline: 3.0486x; 1.0321x over previous
import numpy as np
import jax
import jax.numpy as jnp
from jax.experimental import pallas as pl
from jax.experimental.pallas import tpu as pltpu


def _pts_table_t():
    vert_angles = np.radians(np.concatenate((
        np.linspace(4 + 1.0 / 3, -8 - 1.0 / 3, 40),
        np.linspace(-8 - 1.0 / 3 - 1.0 / 2, -24 - 1.0 / 3, 32))))
    hor_angles = np.radians(np.flip(np.arange(0, 360, 0.1728)) + 180)
    ray = np.array([1.0, 0, 0])
    vert_rotmat = np.array([[[np.cos(a), 0, -np.sin(a)], [0, 1, 0],
                             [np.sin(a), 0, np.cos(a)]] for a in vert_angles])
    hor_rotmat = np.array([[[np.cos(a), -np.sin(a), 0],
                            [np.sin(a), np.cos(a), 0],
                            [0, 0, 1]] for a in hor_angles])
    v = vert_rotmat @ ray
    pts = np.einsum('xij,yj->iyx', hor_rotmat, v)
    return pts.astype(np.float32)


_PTS_T = _pts_table_t()


_BB = 8


def _xyz_kernel(data_ref, pts_ref, out_ref):
    for i in range(_BB):
        dist = data_ref[i, 0]
        maskv = data_ref[i, 1]
        m = maskv >= 0.5
        zero = jnp.zeros((), dtype=dist.dtype)
        md = jnp.where(m, dist, zero)
        out_ref[i, 0] = md * pts_ref[0]
        out_ref[i, 1] = md * pts_ref[1]
        out_ref[i, 2] = md * pts_ref[2]
        out_ref[i, 3] = maskv


def kernel(data):
    b, c, ys, xs = data.shape
    pts = _PTS_T[:, :ys, :xs]
    return pl.pallas_call(
        _xyz_kernel,
        grid=(b // _BB,),
        in_specs=[
            pl.BlockSpec((_BB, c, ys, xs), lambda i: (i, 0, 0, 0)),
            pl.BlockSpec((3, ys, xs), lambda i: (0, 0, 0)),
        ],
        out_specs=pl.BlockSpec((_BB, 4, ys, xs), lambda i: (i, 0, 0, 0)),
        out_shape=jax.ShapeDtypeStruct((b, 4, ys, xs), data.dtype),
        compiler_params=pltpu.CompilerParams(
            vmem_limit_bytes=100 * 1024 * 1024,
        ),
    )(data, pts)
